# baseline (speedup 1.0000x reference)
import functools

import jax
import jax.numpy as jnp
from jax import lax
from jax.experimental import pallas as pl
from jax.experimental.pallas import tpu as pltpu
from jax.experimental.pallas import tpu_sc as plsc

EPS = 1e-6
NC = 2
NS = 16
NW = NC * NS
L = 16

CHUNK = 1024
DMA_ROWS = 128


def _rsqrt(v):
    i = plsc.bitcast(v, jnp.int32)
    y = plsc.bitcast(jnp.int32(0x5F3759DF) - (i >> 1), jnp.float32)
    for _ in range(3):
        y = y * (1.5 - (0.5 * v) * y * y)
    return y


def _make_kernel(n_rows, hidden):
    assert hidden == 4 * L
    rows_per_w = n_rows // NW
    assert rows_per_w * NW == n_rows
    n_chunks = rows_per_w // CHUNK
    assert n_chunks * CHUNK == rows_per_w

    mesh = plsc.VectorSubcoreMesh(core_axis_name="c", subcore_axis_name="s")

    @functools.partial(
        pl.kernel,
        out_type=jax.ShapeDtypeStruct((n_rows, hidden), jnp.float32),
        mesh=mesh,
        scratch_types=[
            pltpu.VMEM((CHUNK,), jnp.int32),
            pltpu.VMEM((CHUNK, hidden), jnp.float32),
            pltpu.VMEM((hidden,), jnp.float32),
            pltpu.VMEM((hidden,), jnp.float32),
            pltpu.SemaphoreType.DMA,
        ],
        compiler_params=pltpu.CompilerParams(
            needs_layout_passes=False, use_tc_tiling_on_sc=False),
    )
    def kern(x_ref, tab_ref, al_ref, be_ref, out_ref, idx_v, rows_v, al_v, be_v, sem):
        wid = lax.axis_index("s") * NC + lax.axis_index("c")
        pltpu.sync_copy(al_ref, al_v)
        pltpu.sync_copy(be_ref, be_v)

        def chunk_body(ci, carry):
            base = wid * rows_per_w + ci * CHUNK
            pltpu.sync_copy(x_ref.at[pl.ds(base, CHUNK)], idx_v)
            handles = []
            for j in range(CHUNK // DMA_ROWS):
                handles.append(pltpu.async_copy(
                    tab_ref.at[idx_v.at[pl.ds(j * DMA_ROWS, DMA_ROWS)]],
                    rows_v.at[pl.ds(j * DMA_ROWS, DMA_ROWS)],
                    sem,
                ))
            for h in handles:
                h.wait()

            def group_body(g, carry2):
                rvec = g * L + lax.iota(jnp.int32, L)
                s = jnp.zeros((L,), jnp.float32)
                ss = jnp.zeros((L,), jnp.float32)
                for j in range(hidden):
                    jvec = jnp.full((L,), j, jnp.int32)
                    c = plsc.load_gather(rows_v, [rvec, jvec])
                    s = s + c
                    ss = ss + c * c
                mean = s * (1.0 / hidden)
                var = jnp.maximum((ss - s * mean) * (1.0 / (hidden - 1)),
                                  jnp.float32(0.0))
                std = var * _rsqrt(var)
                inv = 1.0 / (std + EPS)
                for k in range(hidden // L):
                    a_vec = al_v[pl.ds(k * L, L)]
                    b_vec = be_v[pl.ds(k * L, L)]
                    for jj in range(L):
                        j = k * L + jj
                        jvec = jnp.full((L,), j, jnp.int32)
                        c = plsc.load_gather(rows_v, [rvec, jvec])
                        o = a_vec[jj] * ((c - mean) * inv + b_vec[jj])
                        plsc.store_scatter(rows_v, [rvec, jvec], o)
                return carry2

            lax.fori_loop(0, CHUNK // L, group_body, 0)
            pltpu.sync_copy(rows_v, out_ref.at[pl.ds(base, CHUNK)])
            return carry

        lax.fori_loop(0, n_chunks, chunk_body, 0)

    return kern


def kernel(x, table, alpha, beta):
    b, l = x.shape
    vocab, hidden = table.shape
    x_flat = x.reshape(-1).astype(jnp.int32)
    kern = _make_kernel(b * l, hidden)
    out = kern(x_flat, table, alpha, beta)
    return out.reshape(b, l, hidden)


# speedup vs baseline: 1.7245x; 1.7245x over previous
import functools

import jax
import jax.numpy as jnp
from jax import lax
from jax.experimental import pallas as pl
from jax.experimental.pallas import tpu as pltpu
from jax.experimental.pallas import tpu_sc as plsc

EPS = 1e-6
NC = 2
NS = 16
NW = NC * NS
L = 16

CHUNK = 256
DMA_ROWS = 128


def _rsqrt(v):
    i = plsc.bitcast(v, jnp.int32)
    y = plsc.bitcast(jnp.int32(0x5F3759DF) - (i >> 1), jnp.float32)
    for _ in range(3):
        y = y * (1.5 - (0.5 * v) * y * y)
    return y


def _make_kernel(n_rows, hidden):
    assert hidden == 4 * L
    rows_per_w = n_rows // NW
    assert rows_per_w * NW == n_rows
    n_chunks = rows_per_w // CHUNK
    assert n_chunks * CHUNK == rows_per_w

    mesh = plsc.VectorSubcoreMesh(core_axis_name="c", subcore_axis_name="s")

    @functools.partial(
        pl.kernel,
        out_type=jax.ShapeDtypeStruct((n_rows, hidden), jnp.float32),
        mesh=mesh,
        scratch_types=[
            pltpu.VMEM((CHUNK,), jnp.int32),
            pltpu.VMEM((CHUNK, 2 * hidden), jnp.float32),
            pltpu.VMEM((CHUNK, hidden), jnp.float32),
            pltpu.VMEM((hidden,), jnp.float32),
            pltpu.VMEM((hidden,), jnp.float32),
            pltpu.SemaphoreType.DMA,
        ],
        compiler_params=pltpu.CompilerParams(needs_layout_passes=False),
    )
    def kern(x_ref, tab_ref, al_ref, be_ref, out_ref, idx_v, rows_v, out_v, al_v, be_v, sem):
        wid = lax.axis_index("s") * NC + lax.axis_index("c")
        pltpu.sync_copy(al_ref, al_v)
        pltpu.sync_copy(be_ref, be_v)
        a_vecs = [al_v[pl.ds(k * L, L)] for k in range(hidden // L)]
        b_vecs = [be_v[pl.ds(k * L, L)] for k in range(hidden // L)]

        def chunk_body(ci, carry):
            base = wid * rows_per_w + ci * CHUNK
            pltpu.sync_copy(x_ref.at[pl.ds(base, CHUNK)], idx_v)
            handles = []
            for j in range(CHUNK // DMA_ROWS):
                handles.append(pltpu.async_copy(
                    tab_ref.at[idx_v.at[pl.ds(j * DMA_ROWS, DMA_ROWS)]],
                    rows_v.at[pl.ds(j * DMA_ROWS, DMA_ROWS)],
                    sem,
                ))
            for h in handles:
                h.wait()

            def group_body(g, carry2):
                r0 = g * L
                rvec = r0 + lax.iota(jnp.int32, L)
                s_acc = [jnp.zeros((L,), jnp.float32) for _ in range(4)]
                q_acc = [jnp.zeros((L,), jnp.float32) for _ in range(4)]
                for j in range(hidden):
                    jvec = jnp.full((L,), j, jnp.int32)
                    c = plsc.load_gather(rows_v, [rvec, jvec])
                    s_acc[j % 4] = s_acc[j % 4] + c
                    q_acc[j % 4] = q_acc[j % 4] + c * c
                s = (s_acc[0] + s_acc[1]) + (s_acc[2] + s_acc[3])
                ss = (q_acc[0] + q_acc[1]) + (q_acc[2] + q_acc[3])
                mean = s * (1.0 / hidden)
                var = jnp.maximum((ss - s * mean) * (1.0 / (hidden - 1)),
                                  jnp.float32(0.0))
                std = var * _rsqrt(var)
                inv = 1.0 / (std + EPS)
                for l in range(L):
                    m_l = mean[l]
                    i_l = inv[l]
                    for k in range(hidden // L):
                        v = rows_v[r0 + l, pl.ds(k * L, L)]
                        o = a_vecs[k] * ((v - m_l) * i_l + b_vecs[k])
                        out_v[r0 + l, pl.ds(k * L, L)] = o
                return carry2

            lax.fori_loop(0, CHUNK // L, group_body, 0)
            pltpu.sync_copy(out_v, out_ref.at[pl.ds(base, CHUNK)])
            return carry

        lax.fori_loop(0, n_chunks, chunk_body, 0)

    return kern


def kernel(x, table, alpha, beta):
    b, l = x.shape
    vocab, hidden = table.shape
    x_flat = x.reshape(-1).astype(jnp.int32)
    table_p = jnp.pad(table, ((0, 0), (0, hidden)))
    kern = _make_kernel(b * l, hidden)
    out = kern(x_flat, table_p, alpha, beta)
    return out.reshape(b, l, hidden)


# speedup vs baseline: 3.5988x; 2.0868x over previous
import functools

import jax
import jax.numpy as jnp
from jax import lax
from jax.experimental import pallas as pl
from jax.experimental.pallas import tpu as pltpu
from jax.experimental.pallas import tpu_sc as plsc

EPS = 1e-6
NC = 2
NS = 16
NW = NC * NS
L = 16

CHUNK = 256
DMA_ROWS = 128
P_PITCH = L + 1
Q_OFF = L * P_PITCH


def _rsqrt(v):
    i = plsc.bitcast(v, jnp.int32)
    y = plsc.bitcast(jnp.int32(0x5F3759DF) - (i >> 1), jnp.float32)
    for _ in range(3):
        y = y * (1.5 - (0.5 * v) * y * y)
    return y


def _make_kernel(n_rows, hidden):
    assert hidden == 4 * L
    rows_per_w = n_rows // NW
    assert rows_per_w * NW == n_rows
    n_chunks = rows_per_w // CHUNK
    assert n_chunks * CHUNK == rows_per_w

    mesh = plsc.VectorSubcoreMesh(core_axis_name="c", subcore_axis_name="s")

    @functools.partial(
        pl.kernel,
        out_type=jax.ShapeDtypeStruct((n_rows, hidden), jnp.float32),
        mesh=mesh,
        scratch_types=[
            pltpu.VMEM((CHUNK,), jnp.int32),
            pltpu.VMEM((CHUNK, 2 * hidden), jnp.float32),
            pltpu.VMEM((CHUNK, hidden), jnp.float32),
            pltpu.VMEM((hidden,), jnp.float32),
            pltpu.VMEM((hidden,), jnp.float32),
            pltpu.VMEM((2 * L * P_PITCH,), jnp.float32),
            pltpu.SemaphoreType.DMA,
        ],
        compiler_params=pltpu.CompilerParams(needs_layout_passes=False),
    )
    def kern(x_ref, tab_ref, al_ref, be_ref, out_ref, idx_v, rows_v, out_v, al_v, be_v, p_v, sem):
        wid = lax.axis_index("s") * NC + lax.axis_index("c")
        pltpu.sync_copy(al_ref, al_v)
        pltpu.sync_copy(be_ref, be_v)
        a_vecs = [al_v[pl.ds(k * L, L)] for k in range(hidden // L)]
        b_vecs = [be_v[pl.ds(k * L, L)] for k in range(hidden // L)]

        def chunk_body(ci, carry):
            base = wid * rows_per_w + ci * CHUNK
            pltpu.sync_copy(x_ref.at[pl.ds(base, CHUNK)], idx_v)
            handles = []
            for j in range(CHUNK // DMA_ROWS):
                handles.append(pltpu.async_copy(
                    tab_ref.at[idx_v.at[pl.ds(j * DMA_ROWS, DMA_ROWS)]],
                    rows_v.at[pl.ds(j * DMA_ROWS, DMA_ROWS)],
                    sem,
                ))
            for h in handles:
                h.wait()

            iota = lax.iota(jnp.int32, L)
            iota_p = iota * P_PITCH

            def group_body(g, carry2):
                r0 = g * L
                for l in range(L):
                    v = [rows_v[r0 + l, pl.ds(k * L, L)]
                         for k in range(hidden // L)]
                    s_l = (v[0] + v[1]) + (v[2] + v[3])
                    q_l = (v[0] * v[0] + v[1] * v[1]) + (v[2] * v[2] + v[3] * v[3])
                    sidx = iota + (P_PITCH * l)
                    plsc.store_scatter(p_v, [sidx], s_l)
                    plsc.store_scatter(p_v, [sidx + Q_OFF], q_l)
                s_acc = [jnp.zeros((L,), jnp.float32) for _ in range(4)]
                q_acc = [jnp.zeros((L,), jnp.float32) for _ in range(4)]
                for c in range(L):
                    gv = plsc.load_gather(p_v, [iota_p + c])
                    hv = plsc.load_gather(p_v, [iota_p + (Q_OFF + c)])
                    s_acc[c % 4] = s_acc[c % 4] + gv
                    q_acc[c % 4] = q_acc[c % 4] + hv
                s = (s_acc[0] + s_acc[1]) + (s_acc[2] + s_acc[3])
                ss = (q_acc[0] + q_acc[1]) + (q_acc[2] + q_acc[3])
                mean = s * (1.0 / hidden)
                var = jnp.maximum((ss - s * mean) * (1.0 / (hidden - 1)),
                                  jnp.float32(0.0))
                std = var * _rsqrt(var)
                inv = 1.0 / (std + EPS)
                for l in range(L):
                    m_l = mean[l]
                    i_l = inv[l]
                    for k in range(hidden // L):
                        v = rows_v[r0 + l, pl.ds(k * L, L)]
                        o = a_vecs[k] * ((v - m_l) * i_l + b_vecs[k])
                        out_v[r0 + l, pl.ds(k * L, L)] = o
                return carry2

            lax.fori_loop(0, CHUNK // L, group_body, 0)
            pltpu.sync_copy(out_v, out_ref.at[pl.ds(base, CHUNK)])
            return carry

        lax.fori_loop(0, n_chunks, chunk_body, 0)

    return kern


def kernel(x, table, alpha, beta):
    b, l = x.shape
    vocab, hidden = table.shape
    x_flat = x.reshape(-1).astype(jnp.int32)
    table_p = jnp.pad(table, ((0, 0), (0, hidden)))
    kern = _make_kernel(b * l, hidden)
    out = kern(x_flat, table_p, alpha, beta)
    return out.reshape(b, l, hidden)
